# baseline (speedup 1.0000x reference)
import functools

import jax
import jax.numpy as jnp
from jax import lax
from jax.experimental import pallas as pl
from jax.experimental.pallas import tpu as pltpu
from jax.experimental.pallas import tpu_sc as plsc

NUM_CLASSES = 100000
FEAT_DIM = 64
BATCH = 16384

_INFO = plsc.get_sparse_core_info()
_NC = _INFO.num_cores
_NS = _INFO.num_subcores
_NW = _NC * _NS
_L = _INFO.num_lanes

_B_PER_W = BATCH // _NW
_CHUNK = 128
_NCHUNK = _B_PER_W // _CHUNK
_PAIR = 2 * FEAT_DIM

_BLK = 512
_FULL_BLOCKS = NUM_CLASSES // _BLK
_EXTRA = _FULL_BLOCKS - 6 * _NW
_TAIL32 = NUM_CLASSES - _FULL_BLOCKS * _BLK - 128
_NPAIR = NUM_CLASSES // 2


def _fmt_transpose(slab, obuf, ngroups):
    lane = lax.iota(jnp.int32, _L)
    half = (lane & 1) * FEAT_DIM

    def g_body(g, _):
        row = g * (_L // 2) + lax.shift_right_logical(lane, 1)
        for f in range(FEAT_DIM):
            v = slab[f, pl.ds(pl.multiple_of(g * _L, _L), _L)]
            plsc.store_scatter(obuf, [row, half + f], v)
        return 0

    lax.fori_loop(0, ngroups, g_body, 0)


def _fmt_body(ct_hbm, xt_hbm, tail2_hbm, cen2_hbm, x2_hbm, slab_v, obuf_v,
              tail_v, sem):
    wid = lax.axis_index("s") * _NC + lax.axis_index("c")

    pltpu.sync_copy(xt_hbm.at[:, pl.ds(wid * _B_PER_W, _BLK)], slab_v)
    _fmt_transpose(slab_v, obuf_v, _BLK // _L)
    pltpu.sync_copy(obuf_v, x2_hbm.at[pl.ds(wid * (_BLK // 2), _BLK // 2)])

    nblk = jnp.where(wid < _EXTRA, 7, 6)
    c0 = wid * (6 * _BLK) + jnp.minimum(wid, _EXTRA) * _BLK

    def blk_body(b, _):
        cb = c0 + b * _BLK
        pltpu.sync_copy(ct_hbm.at[:, pl.ds(cb, _BLK)], slab_v)
        _fmt_transpose(slab_v, obuf_v, _BLK // _L)
        pltpu.sync_copy(
            obuf_v,
            cen2_hbm.at[pl.ds(lax.div(cb, 2), _BLK // 2)])
        return 0

    lax.fori_loop(0, nblk, blk_body, 0)

    @pl.when(wid == _NW - 1)
    def _():
        tb = _FULL_BLOCKS * _BLK
        pltpu.sync_copy(ct_hbm.at[:, pl.ds(tb, _PAIR)],
                        slab_v.at[:, pl.ds(0, _PAIR)])
        _fmt_transpose(slab_v, obuf_v, _PAIR // _L)
        pltpu.sync_copy(obuf_v.at[pl.ds(0, _PAIR // 2)],
                        cen2_hbm.at[pl.ds(tb // 2, _PAIR // 2)])
        pltpu.sync_copy(tail2_hbm, tail_v)
        pltpu.sync_copy(tail_v,
                        cen2_hbm.at[pl.ds((tb + _PAIR) // 2, _TAIL32 // 2)])


def _loss_body(x_hbm, lab_hbm, cen_hbm, out_hbm, idx_v, lab_s, x_v, c_v,
               acc_v, s0, s1, s2, s3, xsem):
    wid = lax.axis_index("s") * _NC + lax.axis_index("c")
    base = wid * _NCHUNK

    pltpu.sync_copy(lab_hbm.at[pl.ds(base, _NCHUNK)], idx_v)
    xcopy = pltpu.async_copy(
        x_hbm.at[pl.ds(base * (_CHUNK // 2), _B_PER_W // 2)], x_v, xsem)

    for j in range(_NCHUNK):
        for v in range(_CHUNK // _L):
            sl = pl.ds(v * _L, _L)
            lab = idx_v[j, sl]
            lab_s[j, sl] = (lab & 1) * FEAT_DIM
            idx_v[j, sl] = lax.shift_right_logical(lab, 1)

    sems = (s0, s1, s2, s3)
    gathers = [
        pltpu.async_copy(cen_hbm.at[idx_v.at[j]], c_v.at[j], sems[j])
        for j in range(_NCHUNK)
    ]
    xcopy.wait()

    def make_group_body(j):
        def group_body(k, accs):
            a0, a1, a2, a3 = accs
            offs = lab_s[j, pl.ds(k * _L, _L)]
            for t in range(_L):
                s = k * _L + t
                xr = j * (_CHUNK // 2) + lax.div(s, 2)
                xoff = FEAT_DIM * (t % 2)
                coff = pl.multiple_of(offs[t], FEAT_DIM)
                d0 = (x_v[xr, pl.ds(xoff, _L)]
                      - c_v[j, s, pl.ds(coff, _L)])
                d1 = (x_v[xr, pl.ds(xoff + _L, _L)]
                      - c_v[j, s, pl.ds(coff + _L, _L)])
                d2 = (x_v[xr, pl.ds(xoff + 2 * _L, _L)]
                      - c_v[j, s, pl.ds(coff + 2 * _L, _L)])
                d3 = (x_v[xr, pl.ds(xoff + 3 * _L, _L)]
                      - c_v[j, s, pl.ds(coff + 3 * _L, _L)])
                a0 = a0 + d0 * d0
                a1 = a1 + d1 * d1
                a2 = a2 + d2 * d2
                a3 = a3 + d3 * d3
            return (a0, a1, a2, a3)
        return group_body

    zero = jnp.zeros((_L,), jnp.float32)
    accs = (zero, zero, zero, zero)
    for j in range(_NCHUNK):
        gathers[j].wait()
        accs = lax.fori_loop(0, _CHUNK // _L, make_group_body(j), accs)

    acc_v[...] = accs[0] + accs[1] + accs[2] + accs[3]
    pltpu.sync_copy(acc_v, out_hbm.at[wid])


@jax.jit
def _center_loss(x, labels, centers):
    xt = x.T
    ct = centers.T
    tail2 = centers[NUM_CLASSES - _TAIL32:].reshape(_TAIL32 // 2, _PAIR)
    lab = labels.astype(jnp.int32).reshape(_NW * _NCHUNK, _CHUNK)

    mesh = plsc.VectorSubcoreMesh(core_axis_name="c", subcore_axis_name="s")
    params = pltpu.CompilerParams(use_tc_tiling_on_sc=True)

    fmt = functools.partial(
        pl.kernel,
        out_type=(
            jax.ShapeDtypeStruct((_NPAIR, _PAIR), jnp.float32),
            jax.ShapeDtypeStruct((BATCH // 2, _PAIR), jnp.float32),
        ),
        mesh=mesh,
        compiler_params=pltpu.CompilerParams(
            use_tc_tiling_on_sc=True, needs_layout_passes=False),
        scratch_types=[
            pltpu.VMEM((FEAT_DIM, _BLK), jnp.float32),
            pltpu.VMEM((_BLK // 2, _PAIR), jnp.float32),
            pltpu.VMEM((_TAIL32 // 2, _PAIR), jnp.float32),
            pltpu.SemaphoreType.DMA,
        ],
    )(_fmt_body)
    cen2, x2 = fmt(ct, xt, tail2)

    loss = functools.partial(
        pl.kernel,
        out_type=jax.ShapeDtypeStruct((_NW, _L), jnp.float32),
        mesh=mesh,
        compiler_params=params,
        scratch_types=[
            pltpu.VMEM((_NCHUNK, _CHUNK), jnp.int32),
            pltpu.VMEM((_NCHUNK, _CHUNK), jnp.int32),
            pltpu.VMEM((_B_PER_W // 2, _PAIR), jnp.float32),
            pltpu.VMEM((_NCHUNK, _CHUNK, _PAIR), jnp.float32),
            pltpu.VMEM((_L,), jnp.float32),
            pltpu.SemaphoreType.DMA,
            pltpu.SemaphoreType.DMA,
            pltpu.SemaphoreType.DMA,
            pltpu.SemaphoreType.DMA,
            pltpu.SemaphoreType.DMA,
        ],
    )(_loss_body)
    partials = loss(x2, lab, cen2)
    return jnp.sum(partials) / 2.0 / BATCH


def kernel(x, labels, centers):
    return _center_loss(x, labels, centers)


# speedup vs baseline: 1.9630x; 1.9630x over previous
import functools

import jax
import jax.numpy as jnp
from jax import lax
from jax.experimental import pallas as pl
from jax.experimental.pallas import tpu as pltpu
from jax.experimental.pallas import tpu_sc as plsc

NUM_CLASSES = 100000
FEAT_DIM = 64
BATCH = 16384

_INFO = plsc.get_sparse_core_info()
_NC = _INFO.num_cores
_NS = _INFO.num_subcores
_NW = _NC * _NS
_L = _INFO.num_lanes

_B_PER_W = BATCH // _NW
_CHUNK = 128
_NCHUNK = _B_PER_W // _CHUNK
_PAIR = 2 * FEAT_DIM

_CSPLIT = 51200
_XSPLIT = BATCH // 2
_BLK = 512


def _pack_body(a_ref, b_ref, out_ref):
    out_ref[...] = jnp.concatenate([a_ref[...].T, b_ref[...].T], axis=1)


def _tc_pack(src_t, split, blk):
    nblk = split // blk
    last = (src_t.shape[1] + blk - 1) // blk - 1
    return pl.pallas_call(
        _pack_body,
        grid=(nblk,),
        in_specs=[
            pl.BlockSpec((FEAT_DIM, blk), lambda i: (0, i)),
            pl.BlockSpec(
                (FEAT_DIM, blk),
                lambda i, n=nblk, m=last: (0, jnp.minimum(i + n, m))),
        ],
        out_specs=pl.BlockSpec((blk, _PAIR), lambda i: (i, 0)),
        out_shape=jax.ShapeDtypeStruct((split, _PAIR), jnp.float32),
    )(src_t, src_t)


def _loss_body(x_hbm, lab_hbm, cen_hbm, out_hbm, idx_v, lab_s, x_v, c_v,
               acc_v, s0, s1, s2, s3, x0, x1):
    wid = lax.axis_index("s") * _NC + lax.axis_index("c")
    base = wid * _NCHUNK
    xrow = lax.rem(wid, _NS) * _B_PER_W
    xcol = lax.div(wid, _NS) * FEAT_DIM

    pltpu.sync_copy(lab_hbm.at[pl.ds(base, _NCHUNK)], idx_v)

    for j in range(_NCHUNK):
        for v in range(_CHUNK // _L):
            sl = pl.ds(v * _L, _L)
            lab = idx_v[j, sl]
            hi = jnp.where(lab >= _CSPLIT, 1, 0)
            lab_s[j, sl] = hi * FEAT_DIM
            idx_v[j, sl] = lab - hi * _CSPLIT

    xsems = (x0, x1)
    xcopies = [
        pltpu.async_copy(
            x_hbm.at[pl.ds(xrow + j * _CHUNK, _CHUNK)], x_v.at[j % 2],
            xsems[j % 2])
        for j in range(2)
    ]
    sems = (s0, s1, s2, s3)
    gathers = [
        pltpu.async_copy(cen_hbm.at[idx_v.at[j]], c_v.at[j], sems[j])
        for j in range(_NCHUNK)
    ]

    def make_group_body(j):
        def group_body(k, accs):
            a0, a1, a2, a3 = accs
            offs = lab_s[j, pl.ds(k * _L, _L)]
            for t in range(_L):
                s = k * _L + t
                coff = pl.multiple_of(offs[t], FEAT_DIM)
                xo = pl.multiple_of(xcol, FEAT_DIM)
                d0 = (x_v[j % 2, s, pl.ds(xo, _L)]
                      - c_v[j, s, pl.ds(coff, _L)])
                d1 = (x_v[j % 2, s, pl.ds(xo + _L, _L)]
                      - c_v[j, s, pl.ds(coff + _L, _L)])
                d2 = (x_v[j % 2, s, pl.ds(xo + 2 * _L, _L)]
                      - c_v[j, s, pl.ds(coff + 2 * _L, _L)])
                d3 = (x_v[j % 2, s, pl.ds(xo + 3 * _L, _L)]
                      - c_v[j, s, pl.ds(coff + 3 * _L, _L)])
                a0 = a0 + d0 * d0
                a1 = a1 + d1 * d1
                a2 = a2 + d2 * d2
                a3 = a3 + d3 * d3
            return (a0, a1, a2, a3)
        return group_body

    zero = jnp.zeros((_L,), jnp.float32)
    accs = (zero, zero, zero, zero)
    for j in range(_NCHUNK):
        xcopies[j].wait()
        gathers[j].wait()
        accs = lax.fori_loop(0, _CHUNK // _L, make_group_body(j), accs)
        if j + 2 < _NCHUNK:
            xcopies.append(pltpu.async_copy(
                x_hbm.at[pl.ds(xrow + (j + 2) * _CHUNK, _CHUNK)],
                x_v.at[j % 2], xsems[j % 2]))

    acc_v[...] = accs[0] + accs[1] + accs[2] + accs[3]
    pltpu.sync_copy(acc_v, out_hbm.at[wid])


@jax.jit
def _center_loss(x, labels, centers):
    xt = x.T
    ct = centers.T
    lab = labels.astype(jnp.int32).reshape(_NW * _NCHUNK, _CHUNK)

    cen2 = _tc_pack(ct, _CSPLIT, _BLK)
    x2 = _tc_pack(xt, _XSPLIT, _BLK)

    loss = functools.partial(
        pl.kernel,
        out_type=jax.ShapeDtypeStruct((_NW, _L), jnp.float32),
        mesh=plsc.VectorSubcoreMesh(core_axis_name="c", subcore_axis_name="s"),
        compiler_params=pltpu.CompilerParams(use_tc_tiling_on_sc=True),
        scratch_types=[
            pltpu.VMEM((_NCHUNK, _CHUNK), jnp.int32),
            pltpu.VMEM((_NCHUNK, _CHUNK), jnp.int32),
            pltpu.VMEM((2, _CHUNK, _PAIR), jnp.float32),
            pltpu.VMEM((_NCHUNK, _CHUNK, _PAIR), jnp.float32),
            pltpu.VMEM((_L,), jnp.float32),
            pltpu.SemaphoreType.DMA,
            pltpu.SemaphoreType.DMA,
            pltpu.SemaphoreType.DMA,
            pltpu.SemaphoreType.DMA,
            pltpu.SemaphoreType.DMA,
            pltpu.SemaphoreType.DMA,
        ],
    )(_loss_body)
    partials = loss(x2, lab, cen2)
    return jnp.sum(partials) / 2.0 / BATCH


def kernel(x, labels, centers):
    return _center_loss(x, labels, centers)


# speedup vs baseline: 3.1142x; 1.5865x over previous
import functools

import jax
import jax.numpy as jnp
from jax import lax
from jax.experimental import pallas as pl
from jax.experimental.pallas import tpu as pltpu
from jax.experimental.pallas import tpu_sc as plsc

NUM_CLASSES = 100000
FEAT_DIM = 64
BATCH = 16384

_INFO = plsc.get_sparse_core_info()
_NC = _INFO.num_cores
_NS = _INFO.num_subcores
_NW = _NC * _NS
_L = _INFO.num_lanes

_B_PER_W = BATCH // _NW
_CHUNK = 128
_NCHUNK = _B_PER_W // _CHUNK
_PAIR = 2 * FEAT_DIM

_CSPLIT = 51200
_XSPLIT = BATCH // 2
_BLK = 2048


def _pack_body(a_ref, b_ref, out_ref):
    out_ref[...] = jnp.concatenate([a_ref[...].T, b_ref[...].T], axis=1)


def _tc_pack(src_t, split, blk):
    nblk = split // blk
    last = (src_t.shape[1] + blk - 1) // blk - 1
    return pl.pallas_call(
        _pack_body,
        grid=(nblk,),
        in_specs=[
            pl.BlockSpec((FEAT_DIM, blk), lambda i: (0, i)),
            pl.BlockSpec(
                (FEAT_DIM, blk),
                lambda i, n=nblk, m=last: (0, jnp.minimum(i + n, m))),
        ],
        out_specs=pl.BlockSpec((blk, _PAIR), lambda i: (i, 0)),
        out_shape=jax.ShapeDtypeStruct((split, _PAIR), jnp.float32),
    )(src_t, src_t)


def _loss_body(x_hbm, lab_hbm, cen_hbm, out_hbm, idx_v, lab_s, x_v, c_v,
               acc_v, s0, s1, s2, s3, x0, x1):
    wid = lax.axis_index("s") * _NC + lax.axis_index("c")
    base = wid * _NCHUNK
    xrow = lax.rem(wid, _NS) * _B_PER_W
    xcol = lax.div(wid, _NS) * FEAT_DIM

    pltpu.sync_copy(lab_hbm.at[pl.ds(base, _NCHUNK)], idx_v)

    for j in range(_NCHUNK):
        for v in range(_CHUNK // _L):
            sl = pl.ds(v * _L, _L)
            lab = idx_v[j, sl]
            hi = jnp.where(lab >= _CSPLIT, 1, 0)
            lab_s[j, sl] = hi * FEAT_DIM
            idx_v[j, sl] = lab - hi * _CSPLIT

    xsems = (x0, x1)
    xcopies = [
        pltpu.async_copy(
            x_hbm.at[pl.ds(xrow + j * _CHUNK, _CHUNK)], x_v.at[j % 2],
            xsems[j % 2])
        for j in range(2)
    ]
    sems = (s0, s1, s2, s3)
    gathers = [
        pltpu.async_copy(cen_hbm.at[idx_v.at[j]], c_v.at[j], sems[j])
        for j in range(_NCHUNK)
    ]

    def make_group_body(j):
        def group_body(k, accs):
            a0, a1, a2, a3 = accs
            offs = lab_s[j, pl.ds(k * _L, _L)]
            for t in range(_L):
                s = k * _L + t
                coff = pl.multiple_of(offs[t], FEAT_DIM)
                xo = pl.multiple_of(xcol, FEAT_DIM)
                d0 = (x_v[j % 2, s, pl.ds(xo, _L)]
                      - c_v[j, s, pl.ds(coff, _L)])
                d1 = (x_v[j % 2, s, pl.ds(xo + _L, _L)]
                      - c_v[j, s, pl.ds(coff + _L, _L)])
                d2 = (x_v[j % 2, s, pl.ds(xo + 2 * _L, _L)]
                      - c_v[j, s, pl.ds(coff + 2 * _L, _L)])
                d3 = (x_v[j % 2, s, pl.ds(xo + 3 * _L, _L)]
                      - c_v[j, s, pl.ds(coff + 3 * _L, _L)])
                a0 = a0 + d0 * d0
                a1 = a1 + d1 * d1
                a2 = a2 + d2 * d2
                a3 = a3 + d3 * d3
            return (a0, a1, a2, a3)
        return group_body

    zero = jnp.zeros((_L,), jnp.float32)
    accs = (zero, zero, zero, zero)
    for j in range(_NCHUNK):
        xcopies[j].wait()
        gathers[j].wait()
        accs = lax.fori_loop(0, _CHUNK // _L, make_group_body(j), accs)
        if j + 2 < _NCHUNK:
            xcopies.append(pltpu.async_copy(
                x_hbm.at[pl.ds(xrow + (j + 2) * _CHUNK, _CHUNK)],
                x_v.at[j % 2], xsems[j % 2]))

    acc_v[...] = accs[0] + accs[1] + accs[2] + accs[3]
    pltpu.sync_copy(acc_v, out_hbm.at[wid])


@jax.jit
def _center_loss(x, labels, centers):
    xt = x.T
    ct = centers.T
    lab = labels.astype(jnp.int32).reshape(_NW * _NCHUNK, _CHUNK)

    cen2 = _tc_pack(ct, _CSPLIT, _BLK)
    x2 = _tc_pack(xt, _XSPLIT, _BLK)

    loss = functools.partial(
        pl.kernel,
        out_type=jax.ShapeDtypeStruct((_NW, _L), jnp.float32),
        mesh=plsc.VectorSubcoreMesh(core_axis_name="c", subcore_axis_name="s"),
        compiler_params=pltpu.CompilerParams(use_tc_tiling_on_sc=True),
        scratch_types=[
            pltpu.VMEM((_NCHUNK, _CHUNK), jnp.int32),
            pltpu.VMEM((_NCHUNK, _CHUNK), jnp.int32),
            pltpu.VMEM((2, _CHUNK, _PAIR), jnp.float32),
            pltpu.VMEM((_NCHUNK, _CHUNK, _PAIR), jnp.float32),
            pltpu.VMEM((_L,), jnp.float32),
            pltpu.SemaphoreType.DMA,
            pltpu.SemaphoreType.DMA,
            pltpu.SemaphoreType.DMA,
            pltpu.SemaphoreType.DMA,
            pltpu.SemaphoreType.DMA,
            pltpu.SemaphoreType.DMA,
        ],
    )(_loss_body)
    partials = loss(x2, lab, cen2)
    return jnp.sum(partials) / 2.0 / BATCH


def kernel(x, labels, centers):
    return _center_loss(x, labels, centers)


# speedup vs baseline: 3.3288x; 1.0689x over previous
import functools

import jax
import jax.numpy as jnp
from jax import lax
from jax.experimental import pallas as pl
from jax.experimental.pallas import tpu as pltpu
from jax.experimental.pallas import tpu_sc as plsc

NUM_CLASSES = 100000
FEAT_DIM = 64
BATCH = 16384

_INFO = plsc.get_sparse_core_info()
_NC = _INFO.num_cores
_NS = _INFO.num_subcores
_NW = _NC * _NS
_L = _INFO.num_lanes

_B_PER_W = BATCH // _NW
_CHUNK = 128
_NCHUNK = _B_PER_W // _CHUNK
_PAIR = 2 * FEAT_DIM

_CSPLIT = 51200
_XSPLIT = BATCH // 2
_CBLK = 3200
_XBLK = 2048


def _pack_body(a_ref, b_ref, out_ref):
    out_ref[...] = jnp.concatenate([a_ref[...].T, b_ref[...].T], axis=1)


def _tc_pack(src_t, split, blk):
    nblk = split // blk
    last = (src_t.shape[1] + blk - 1) // blk - 1
    return pl.pallas_call(
        _pack_body,
        grid=(nblk,),
        in_specs=[
            pl.BlockSpec((FEAT_DIM, blk), lambda i: (0, i)),
            pl.BlockSpec(
                (FEAT_DIM, blk),
                lambda i, n=nblk, m=last: (0, jnp.minimum(i + n, m))),
        ],
        out_specs=pl.BlockSpec((blk, _PAIR), lambda i: (i, 0)),
        out_shape=jax.ShapeDtypeStruct((split, _PAIR), jnp.float32),
    )(src_t, src_t)


def _loss_body(x_hbm, lab_hbm, cen_hbm, out_hbm, idx_v, lab_s, x_v, c_v,
               acc_v, s0, s1, s2, s3, x0, x1):
    wid = lax.axis_index("s") * _NC + lax.axis_index("c")
    base = wid * _NCHUNK
    xrow = lax.rem(wid, _NS) * _B_PER_W
    xcol = lax.div(wid, _NS) * FEAT_DIM

    pltpu.sync_copy(lab_hbm.at[pl.ds(base, _NCHUNK)], idx_v)

    for j in range(_NCHUNK):
        for v in range(_CHUNK // _L):
            sl = pl.ds(v * _L, _L)
            lab = idx_v[j, sl]
            hi = jnp.where(lab >= _CSPLIT, 1, 0)
            lab_s[j, sl] = hi * FEAT_DIM
            idx_v[j, sl] = lab - hi * _CSPLIT

    xsems = (x0, x1)
    xcopies = [
        pltpu.async_copy(
            x_hbm.at[pl.ds(xrow + j * _CHUNK, _CHUNK)], x_v.at[j % 2],
            xsems[j % 2])
        for j in range(2)
    ]
    sems = (s0, s1, s2, s3)
    gathers = [
        pltpu.async_copy(cen_hbm.at[idx_v.at[j]], c_v.at[j], sems[j])
        for j in range(_NCHUNK)
    ]

    xo = pl.multiple_of(xcol, FEAT_DIM)

    def make_group_body(j):
        def group_body(k, accs):
            a0, a1, a2, a3 = accs
            offs = lab_s[j, pl.ds(k * _L, _L)]
            for t in range(_L):
                s = k * _L + t
                coff = pl.multiple_of(offs[t], FEAT_DIM)
                d0 = (x_v[j % 2, s, pl.ds(xo, _L)]
                      - c_v[j, s, pl.ds(coff, _L)])
                d1 = (x_v[j % 2, s, pl.ds(xo + _L, _L)]
                      - c_v[j, s, pl.ds(coff + _L, _L)])
                d2 = (x_v[j % 2, s, pl.ds(xo + 2 * _L, _L)]
                      - c_v[j, s, pl.ds(coff + 2 * _L, _L)])
                d3 = (x_v[j % 2, s, pl.ds(xo + 3 * _L, _L)]
                      - c_v[j, s, pl.ds(coff + 3 * _L, _L)])
                a0 = a0 + d0 * d0
                a1 = a1 + d1 * d1
                a2 = a2 + d2 * d2
                a3 = a3 + d3 * d3
            return (a0, a1, a2, a3)
        return group_body

    zero = jnp.zeros((_L,), jnp.float32)
    accs = (zero, zero, zero, zero)
    for j in range(_NCHUNK):
        xcopies[j].wait()
        gathers[j].wait()
        accs = lax.fori_loop(0, _CHUNK // _L, make_group_body(j), accs)
        if j + 2 < _NCHUNK:
            xcopies.append(pltpu.async_copy(
                x_hbm.at[pl.ds(xrow + (j + 2) * _CHUNK, _CHUNK)],
                x_v.at[j % 2], xsems[j % 2]))

    acc_v[...] = accs[0] + accs[1] + accs[2] + accs[3]
    pltpu.sync_copy(acc_v, out_hbm.at[wid])


@jax.jit
def _center_loss(x, labels, centers):
    xt = x.T
    ct = centers.T
    lab = labels.astype(jnp.int32).reshape(_NW * _NCHUNK, _CHUNK)

    cen2 = _tc_pack(ct, _CSPLIT, _CBLK)
    x2 = _tc_pack(xt, _XSPLIT, _XBLK)

    loss = functools.partial(
        pl.kernel,
        out_type=jax.ShapeDtypeStruct((_NW, _L), jnp.float32),
        mesh=plsc.VectorSubcoreMesh(core_axis_name="c", subcore_axis_name="s"),
        compiler_params=pltpu.CompilerParams(use_tc_tiling_on_sc=True),
        scratch_types=[
            pltpu.VMEM((_NCHUNK, _CHUNK), jnp.int32),
            pltpu.VMEM((_NCHUNK, _CHUNK), jnp.int32),
            pltpu.VMEM((2, _CHUNK, _PAIR), jnp.float32),
            pltpu.VMEM((_NCHUNK, _CHUNK, _PAIR), jnp.float32),
            pltpu.VMEM((_L,), jnp.float32),
            pltpu.SemaphoreType.DMA,
            pltpu.SemaphoreType.DMA,
            pltpu.SemaphoreType.DMA,
            pltpu.SemaphoreType.DMA,
            pltpu.SemaphoreType.DMA,
            pltpu.SemaphoreType.DMA,
        ],
    )(_loss_body)
    partials = loss(x2, lab, cen2)
    return jnp.sum(partials) / 2.0 / BATCH


def kernel(x, labels, centers):
    return _center_loss(x, labels, centers)


# speedup vs baseline: 3.5274x; 1.0597x over previous
import functools

import jax
import jax.numpy as jnp
from jax import lax
from jax.experimental import pallas as pl
from jax.experimental.pallas import tpu as pltpu
from jax.experimental.pallas import tpu_sc as plsc

NUM_CLASSES = 100000
FEAT_DIM = 64
BATCH = 16384

_INFO = plsc.get_sparse_core_info()
_NC = _INFO.num_cores
_NS = _INFO.num_subcores
_NW = _NC * _NS
_L = _INFO.num_lanes

_B_PER_W = BATCH // _NW
_CHUNK = 128
_NCHUNK = _B_PER_W // _CHUNK
_PAIR = 2 * FEAT_DIM

_CSPLIT = 51200
_XSPLIT = BATCH // 2
_CBLK = 6400
_XBLK = 4096


def _pack_body(a_ref, b_ref, out_ref):
    out_ref[...] = jnp.concatenate([a_ref[...].T, b_ref[...].T], axis=1)


def _tc_pack(src_t, split, blk):
    nblk = split // blk
    last = (src_t.shape[1] + blk - 1) // blk - 1
    return pl.pallas_call(
        _pack_body,
        grid=(nblk,),
        in_specs=[
            pl.BlockSpec((FEAT_DIM, blk), lambda i: (0, i)),
            pl.BlockSpec(
                (FEAT_DIM, blk),
                lambda i, n=nblk, m=last: (0, jnp.minimum(i + n, m))),
        ],
        out_specs=pl.BlockSpec((blk, _PAIR), lambda i: (i, 0)),
        out_shape=jax.ShapeDtypeStruct((split, _PAIR), jnp.float32),
    )(src_t, src_t)


def _loss_body(x_hbm, lab_hbm, cen_hbm, out_hbm, idx_v, lab_s, x_v, c_v,
               acc_v, s0, s1, s2, s3, x0, x1):
    wid = lax.axis_index("s") * _NC + lax.axis_index("c")
    base = wid * _NCHUNK
    xrow = lax.rem(wid, _NS) * _B_PER_W
    xcol = lax.div(wid, _NS) * FEAT_DIM

    pltpu.sync_copy(lab_hbm.at[pl.ds(base, _NCHUNK)], idx_v)

    for j in range(_NCHUNK):
        for v in range(_CHUNK // _L):
            sl = pl.ds(v * _L, _L)
            lab = idx_v[j, sl]
            hi = jnp.where(lab >= _CSPLIT, 1, 0)
            lab_s[j, sl] = hi * FEAT_DIM
            idx_v[j, sl] = lab - hi * _CSPLIT

    xsems = (x0, x1)
    xcopies = [
        pltpu.async_copy(
            x_hbm.at[pl.ds(xrow + j * _CHUNK, _CHUNK)], x_v.at[j % 2],
            xsems[j % 2])
        for j in range(2)
    ]
    sems = (s0, s1, s2, s3)
    gathers = [
        pltpu.async_copy(cen_hbm.at[idx_v.at[j]], c_v.at[j], sems[j])
        for j in range(_NCHUNK)
    ]

    xo = pl.multiple_of(xcol, FEAT_DIM)

    def make_group_body(j):
        def group_body(k, accs):
            a0, a1, a2, a3 = accs
            offs = lab_s[j, pl.ds(k * _L, _L)]
            for t in range(_L):
                s = k * _L + t
                coff = pl.multiple_of(offs[t], FEAT_DIM)
                d0 = (x_v[j % 2, s, pl.ds(xo, _L)]
                      - c_v[j, s, pl.ds(coff, _L)])
                d1 = (x_v[j % 2, s, pl.ds(xo + _L, _L)]
                      - c_v[j, s, pl.ds(coff + _L, _L)])
                d2 = (x_v[j % 2, s, pl.ds(xo + 2 * _L, _L)]
                      - c_v[j, s, pl.ds(coff + 2 * _L, _L)])
                d3 = (x_v[j % 2, s, pl.ds(xo + 3 * _L, _L)]
                      - c_v[j, s, pl.ds(coff + 3 * _L, _L)])
                a0 = a0 + d0 * d0
                a1 = a1 + d1 * d1
                a2 = a2 + d2 * d2
                a3 = a3 + d3 * d3
            return (a0, a1, a2, a3)
        return group_body

    zero = jnp.zeros((_L,), jnp.float32)
    accs = (zero, zero, zero, zero)
    for j in range(_NCHUNK):
        xcopies[j].wait()
        gathers[j].wait()
        accs = lax.fori_loop(0, _CHUNK // _L, make_group_body(j), accs)
        if j + 2 < _NCHUNK:
            xcopies.append(pltpu.async_copy(
                x_hbm.at[pl.ds(xrow + (j + 2) * _CHUNK, _CHUNK)],
                x_v.at[j % 2], xsems[j % 2]))

    acc_v[...] = accs[0] + accs[1] + accs[2] + accs[3]
    pltpu.sync_copy(acc_v, out_hbm.at[wid])


@jax.jit
def _center_loss(x, labels, centers):
    xt = x.T
    ct = centers.T
    lab = labels.astype(jnp.int32).reshape(_NW * _NCHUNK, _CHUNK)

    cen2 = _tc_pack(ct, _CSPLIT, _CBLK)
    x2 = _tc_pack(xt, _XSPLIT, _XBLK)

    loss = functools.partial(
        pl.kernel,
        out_type=jax.ShapeDtypeStruct((_NW, _L), jnp.float32),
        mesh=plsc.VectorSubcoreMesh(core_axis_name="c", subcore_axis_name="s"),
        compiler_params=pltpu.CompilerParams(use_tc_tiling_on_sc=True),
        scratch_types=[
            pltpu.VMEM((_NCHUNK, _CHUNK), jnp.int32),
            pltpu.VMEM((_NCHUNK, _CHUNK), jnp.int32),
            pltpu.VMEM((2, _CHUNK, _PAIR), jnp.float32),
            pltpu.VMEM((_NCHUNK, _CHUNK, _PAIR), jnp.float32),
            pltpu.VMEM((_L,), jnp.float32),
            pltpu.SemaphoreType.DMA,
            pltpu.SemaphoreType.DMA,
            pltpu.SemaphoreType.DMA,
            pltpu.SemaphoreType.DMA,
            pltpu.SemaphoreType.DMA,
            pltpu.SemaphoreType.DMA,
        ],
    )(_loss_body)
    partials = loss(x2, lab, cen2)
    return jnp.sum(partials) / 2.0 / BATCH


def kernel(x, labels, centers):
    return _center_loss(x, labels, centers)
